# Initial kernel scaffold; baseline (speedup 1.0000x reference)
import jax
import jax.numpy as jnp
from jax.experimental import pallas as pl


def kernel(features, anchor_adj, fgp_adj, W1, b1, W2, b2, PW1, pb1, PW2, pb2):
    raise NotImplementedError("write your pallas kernel here")


# speedup vs baseline: 1.8332x; 1.8332x over previous
import jax
import jax.numpy as jnp
from jax.experimental import pallas as pl
from jax.experimental.pallas import tpu as pltpu

_N = 4096
_F32 = jnp.float32
_EOS = 1e-10


def _mm(a, b):
    return jax.lax.dot_general(a, b, (((1,), (0,)), ((), ())),
                               preferred_element_type=_F32)


def _elu1(x):
    return jnp.where(x > 0, x + 1.0, jnp.exp(jnp.minimum(x, 0.0)))


def _s1_body(x_ref, w_ref, b_ref, out_ref):
    out_ref[...] = _mm(x_ref[...], w_ref[...]) + b_ref[...]


def _prop1_body(adj_ref, s_ref, w_ref, b_ref, out_ref):
    h = _mm(adj_ref[...], s_ref[...])
    out_ref[...] = _mm(jnp.maximum(h, 0.0), w_ref[...]) + b_ref[...]


def _prop2_body(adj_ref, t_ref, pw1_ref, pb1_ref, pw2_ref, pb2_ref, zn_ref):
    e = _mm(adj_ref[...], t_ref[...])
    u = jnp.maximum(_mm(e, pw1_ref[...]) + pb1_ref[...], 0.0)
    z = _mm(u, pw2_ref[...]) + pb2_ref[...]
    norm = jnp.sqrt(jnp.sum(z * z, axis=1, keepdims=True))
    zn_ref[...] = z / norm


def _stats_body(fgp_ref, rs_ref, cs_ref):
    i = pl.program_id(0)
    l = _elu1(fgp_ref[...])
    rs_ref[...] = jnp.sum(l, axis=1)[None, :]

    @pl.when(i == 0)
    def _():
        cs_ref[...] = jnp.zeros_like(cs_ref)

    cs_ref[...] += jnp.sum(l, axis=0, keepdims=True)


def _learned_body(rows_ref, cols_ref, rs_ref, cs_ref, rsb_ref, csb_ref,
                  s1_ref, w_ref, b_ref, an_ref, t2_ref):
    inv_row = 1.0 / (jnp.sqrt(0.5 * (rs_ref[...] + cs_ref[...])) + _EOS)
    inv_blk = 1.0 / (jnp.sqrt(0.5 * (rsb_ref[...] + csb_ref[...])) + _EOS)
    inv_col = inv_blk.T
    lr = _elu1(rows_ref[...])
    lc = _elu1(cols_ref[...].T)
    an = (0.5 * inv_col) * (lr + lc) * inv_row
    an_ref[...] = an
    h = _mm(an, s1_ref[...])
    t2_ref[...] = _mm(jnp.maximum(h, 0.0), w_ref[...]) + b_ref[...]


def _sim_body(z1_ref, z2_ref, pos_ref, sr_ref, sc_ref):
    i = pl.program_id(0)
    z1b = z1_ref[...]
    z2 = z2_ref[...]
    s = jax.lax.dot_general(z1b, z2, (((1,), (1,)), ((), ())),
                            preferred_element_type=_F32)
    s = jnp.exp(s * 5.0)
    rb = z1b.shape[0]
    cols = jax.lax.broadcasted_iota(jnp.int32, (rb, _N), 1)
    rows = jax.lax.broadcasted_iota(jnp.int32, (rb, _N), 0) + i * rb
    diag = jnp.where(cols == rows, s, 0.0)
    pos_ref[...] = jnp.sum(diag, axis=1)[None, :]
    sr_ref[...] = jnp.sum(s, axis=1)[None, :]

    @pl.when(i == 0)
    def _():
        sc_ref[...] = jnp.zeros_like(sc_ref)

    sc_ref[...] += jnp.sum(s, axis=0, keepdims=True)


def _loss_body(pos_ref, sr_ref, sc_ref, out_ref):
    pos = pos_ref[...]
    l0 = jnp.log((sc_ref[...] - pos) / pos)
    l1 = jnp.log((sr_ref[...] - pos) / pos)
    val = (0.5 / _N) * (jnp.sum(l0) + jnp.sum(l1))
    out_ref[...] = jnp.broadcast_to(val, out_ref.shape)


def _arb(n):
    return pltpu.CompilerParams(dimension_semantics=("arbitrary",) * n)


def kernel(features, anchor_adj, fgp_adj, W1, b1, W2, b2, PW1, pb1, PW2, pb2):
    b1r = b1.reshape(1, -1)
    b2r = b2.reshape(1, -1)
    pb1r = pb1.reshape(1, -1)
    pb2r = pb2.reshape(1, -1)
    F = features.shape[1]
    H = W1.shape[1]
    R = W2.shape[1]
    P = PW2.shape[1]

    s1 = pl.pallas_call(
        _s1_body,
        out_shape=jax.ShapeDtypeStruct((_N, H), _F32),
    )(features, W1, b1r)

    RB = 512
    grid = (_N // RB,)

    def prop1(adj, s):
        return pl.pallas_call(
            _prop1_body,
            grid=grid,
            in_specs=[
                pl.BlockSpec((RB, _N), lambda i: (i, 0)),
                pl.BlockSpec((_N, H), lambda i: (0, 0)),
                pl.BlockSpec((H, R), lambda i: (0, 0)),
                pl.BlockSpec((1, R), lambda i: (0, 0)),
            ],
            out_specs=pl.BlockSpec((RB, R), lambda i: (i, 0)),
            out_shape=jax.ShapeDtypeStruct((_N, R), _F32),
            compiler_params=_arb(1),
        )(adj, s, W2, b2r)

    def prop2(adj, t):
        return pl.pallas_call(
            _prop2_body,
            grid=grid,
            in_specs=[
                pl.BlockSpec((RB, _N), lambda i: (i, 0)),
                pl.BlockSpec((_N, R), lambda i: (0, 0)),
                pl.BlockSpec((R, P), lambda i: (0, 0)),
                pl.BlockSpec((1, P), lambda i: (0, 0)),
                pl.BlockSpec((P, P), lambda i: (0, 0)),
                pl.BlockSpec((1, P), lambda i: (0, 0)),
            ],
            out_specs=pl.BlockSpec((RB, P), lambda i: (i, 0)),
            out_shape=jax.ShapeDtypeStruct((_N, P), _F32),
            compiler_params=_arb(1),
        )(adj, t, PW1, pb1r, PW2, pb2r)

    t1 = prop1(anchor_adj, s1)
    zn1 = prop2(anchor_adj, t1)

    rs, cs = pl.pallas_call(
        _stats_body,
        grid=grid,
        in_specs=[pl.BlockSpec((RB, _N), lambda i: (i, 0))],
        out_specs=[
            pl.BlockSpec((1, RB), lambda i: (0, i)),
            pl.BlockSpec((1, _N), lambda i: (0, 0)),
        ],
        out_shape=[
            jax.ShapeDtypeStruct((1, _N), _F32),
            jax.ShapeDtypeStruct((1, _N), _F32),
        ],
        compiler_params=_arb(1),
    )(fgp_adj)

    RBL = 256
    an, t2 = pl.pallas_call(
        _learned_body,
        grid=(_N // RBL,),
        in_specs=[
            pl.BlockSpec((RBL, _N), lambda i: (i, 0)),
            pl.BlockSpec((_N, RBL), lambda i: (0, i)),
            pl.BlockSpec((1, _N), lambda i: (0, 0)),
            pl.BlockSpec((1, _N), lambda i: (0, 0)),
            pl.BlockSpec((1, RBL), lambda i: (0, i)),
            pl.BlockSpec((1, RBL), lambda i: (0, i)),
            pl.BlockSpec((_N, H), lambda i: (0, 0)),
            pl.BlockSpec((H, R), lambda i: (0, 0)),
            pl.BlockSpec((1, R), lambda i: (0, 0)),
        ],
        out_specs=[
            pl.BlockSpec((RBL, _N), lambda i: (i, 0)),
            pl.BlockSpec((RBL, R), lambda i: (i, 0)),
        ],
        out_shape=[
            jax.ShapeDtypeStruct((_N, _N), _F32),
            jax.ShapeDtypeStruct((_N, R), _F32),
        ],
        compiler_params=_arb(1),
    )(fgp_adj, fgp_adj, rs, cs, rs, cs, s1, W2, b2r)

    zn2 = prop2(an, t2)

    pos, sr, sc = pl.pallas_call(
        _sim_body,
        grid=grid,
        in_specs=[
            pl.BlockSpec((RB, P), lambda i: (i, 0)),
            pl.BlockSpec((_N, P), lambda i: (0, 0)),
        ],
        out_specs=[
            pl.BlockSpec((1, RB), lambda i: (0, i)),
            pl.BlockSpec((1, RB), lambda i: (0, i)),
            pl.BlockSpec((1, _N), lambda i: (0, 0)),
        ],
        out_shape=[
            jax.ShapeDtypeStruct((1, _N), _F32),
            jax.ShapeDtypeStruct((1, _N), _F32),
            jax.ShapeDtypeStruct((1, _N), _F32),
        ],
        compiler_params=_arb(1),
    )(zn1, zn2)

    lossbuf = pl.pallas_call(
        _loss_body,
        out_shape=jax.ShapeDtypeStruct((1, 128), _F32),
    )(pos, sr, sc)

    return lossbuf[0, 0], an


# speedup vs baseline: 1.9438x; 1.0603x over previous
import jax
import jax.numpy as jnp
from jax.experimental import pallas as pl
from jax.experimental.pallas import tpu as pltpu

_N = 4096
_F32 = jnp.float32
_EOS = 1e-10


def _mm(a, b):
    return jax.lax.dot_general(a, b, (((1,), (0,)), ((), ())),
                               preferred_element_type=_F32)


def _elu1(x):
    return jnp.maximum(x + 1.0, jnp.exp(jnp.minimum(x, 0.0)))


def _k1_body(x_ref, w1_ref, b1_ref, adj_ref, w2_ref, b2_ref,
             t1_ref, s1_ref):
    i = pl.program_id(0)

    @pl.when(i == 0)
    def _():
        s1_ref[...] = _mm(x_ref[...], w1_ref[...]) + b1_ref[...]

    h = _mm(adj_ref[...], s1_ref[...])
    t1_ref[...] = _mm(jnp.maximum(h, 0.0), w2_ref[...]) + b2_ref[...]


def _proj_norm(e, pw1, pb1, pw2, pb2):
    u = jnp.maximum(_mm(e, pw1) + pb1, 0.0)
    z = _mm(u, pw2) + pb2
    return z / jnp.sqrt(jnp.sum(z * z, axis=1, keepdims=True))


def _k2_body(adj_ref, t_ref, pw1_ref, pb1_ref, pw2_ref, pb2_ref, fgp_ref,
             zn_ref, rs_ref, cs_ref):
    i = pl.program_id(0)
    e = _mm(adj_ref[...], t_ref[...])
    zn_ref[...] = _proj_norm(e, pw1_ref[...], pb1_ref[...],
                             pw2_ref[...], pb2_ref[...])
    l = _elu1(fgp_ref[...])
    rs_ref[...] = jnp.sum(l, axis=1)[None, :]

    @pl.when(i == 0)
    def _():
        cs_ref[...] = jnp.zeros_like(cs_ref)

    cs_ref[...] += jnp.sum(l, axis=0, keepdims=True)


def _k3_body(rows_ref, cols_ref, rs_ref, cs_ref, rsb_ref, csb_ref,
             s1_ref, w_ref, b_ref, an_ref, t2_ref):
    inv_row = 1.0 / (jnp.sqrt(0.5 * (rs_ref[...] + cs_ref[...])) + _EOS)
    inv_blk = 1.0 / (jnp.sqrt(0.5 * (rsb_ref[...] + csb_ref[...])) + _EOS)
    inv_col = inv_blk.T
    lr = _elu1(rows_ref[...])
    lc = _elu1(cols_ref[...].T)
    an = (0.5 * inv_col) * (lr + lc) * inv_row
    an_ref[...] = an
    h = _mm(an, s1_ref[...])
    t2_ref[...] = _mm(jnp.maximum(h, 0.0), w_ref[...]) + b_ref[...]


def _k4_body(adj_ref, t_ref, pw1_ref, pb1_ref, pw2_ref, pb2_ref, zn_ref):
    e = _mm(adj_ref[...], t_ref[...])
    zn_ref[...] = _proj_norm(e, pw1_ref[...], pb1_ref[...],
                             pw2_ref[...], pb2_ref[...])


def _k5_body(z1_ref, z2_ref, out_ref, pos_acc, sc_acc, l1_acc):
    i = pl.program_id(0)
    n = pl.num_programs(0)
    z1b = z1_ref[...]
    s = jax.lax.dot_general(z1b, z2_ref[...], (((1,), (1,)), ((), ())),
                            preferred_element_type=_F32)
    s = jnp.exp(s * 5.0)
    rb = z1b.shape[0]
    cols = jax.lax.broadcasted_iota(jnp.int32, (rb, _N), 1)
    rows = jax.lax.broadcasted_iota(jnp.int32, (rb, _N), 0) + i * rb
    pos = jnp.sum(jnp.where(cols == rows, s, 0.0), axis=1)[None, :]
    sr = jnp.sum(s, axis=1)[None, :]

    @pl.when(i == 0)
    def _():
        sc_acc[...] = jnp.zeros_like(sc_acc)
        l1_acc[0, 0] = 0.0

    sc_acc[...] += jnp.sum(s, axis=0, keepdims=True)
    pos_acc[:, pl.ds(i * rb, rb)] = pos
    l1_acc[0, 0] += jnp.sum(jnp.log((sr - pos) / pos))

    @pl.when(i == n - 1)
    def _():
        p = pos_acc[...]
        l0 = jnp.sum(jnp.log((sc_acc[...] - p) / p))
        out_ref[...] = jnp.broadcast_to((0.5 / _N) * (l0 + l1_acc[0, 0]),
                                        out_ref.shape)


def _arb(n):
    return pltpu.CompilerParams(dimension_semantics=("arbitrary",) * n)


def kernel(features, anchor_adj, fgp_adj, W1, b1, W2, b2, PW1, pb1, PW2, pb2):
    b1r = b1.reshape(1, -1)
    b2r = b2.reshape(1, -1)
    pb1r = pb1.reshape(1, -1)
    pb2r = pb2.reshape(1, -1)
    F = features.shape[1]
    H = W1.shape[1]
    R = W2.shape[1]
    P = PW2.shape[1]

    RB = 512
    grid = (_N // RB,)

    t1, s1 = pl.pallas_call(
        _k1_body,
        grid=grid,
        in_specs=[
            pl.BlockSpec((_N, F), lambda i: (0, 0)),
            pl.BlockSpec((F, H), lambda i: (0, 0)),
            pl.BlockSpec((1, H), lambda i: (0, 0)),
            pl.BlockSpec((RB, _N), lambda i: (i, 0)),
            pl.BlockSpec((H, R), lambda i: (0, 0)),
            pl.BlockSpec((1, R), lambda i: (0, 0)),
        ],
        out_specs=[
            pl.BlockSpec((RB, R), lambda i: (i, 0)),
            pl.BlockSpec((_N, H), lambda i: (0, 0)),
        ],
        out_shape=[
            jax.ShapeDtypeStruct((_N, R), _F32),
            jax.ShapeDtypeStruct((_N, H), _F32),
        ],
        compiler_params=_arb(1),
    )(features, W1, b1r, anchor_adj, W2, b2r)

    zn1, rs, cs = pl.pallas_call(
        _k2_body,
        grid=grid,
        in_specs=[
            pl.BlockSpec((RB, _N), lambda i: (i, 0)),
            pl.BlockSpec((_N, R), lambda i: (0, 0)),
            pl.BlockSpec((R, P), lambda i: (0, 0)),
            pl.BlockSpec((1, P), lambda i: (0, 0)),
            pl.BlockSpec((P, P), lambda i: (0, 0)),
            pl.BlockSpec((1, P), lambda i: (0, 0)),
            pl.BlockSpec((RB, _N), lambda i: (i, 0)),
        ],
        out_specs=[
            pl.BlockSpec((RB, P), lambda i: (i, 0)),
            pl.BlockSpec((1, RB), lambda i: (0, i)),
            pl.BlockSpec((1, _N), lambda i: (0, 0)),
        ],
        out_shape=[
            jax.ShapeDtypeStruct((_N, P), _F32),
            jax.ShapeDtypeStruct((1, _N), _F32),
            jax.ShapeDtypeStruct((1, _N), _F32),
        ],
        compiler_params=_arb(1),
    )(anchor_adj, t1, PW1, pb1r, PW2, pb2r, fgp_adj)

    RBL = 256
    an, t2 = pl.pallas_call(
        _k3_body,
        grid=(_N // RBL,),
        in_specs=[
            pl.BlockSpec((RBL, _N), lambda i: (i, 0)),
            pl.BlockSpec((_N, RBL), lambda i: (0, i)),
            pl.BlockSpec((1, _N), lambda i: (0, 0)),
            pl.BlockSpec((1, _N), lambda i: (0, 0)),
            pl.BlockSpec((1, RBL), lambda i: (0, i)),
            pl.BlockSpec((1, RBL), lambda i: (0, i)),
            pl.BlockSpec((_N, H), lambda i: (0, 0)),
            pl.BlockSpec((H, R), lambda i: (0, 0)),
            pl.BlockSpec((1, R), lambda i: (0, 0)),
        ],
        out_specs=[
            pl.BlockSpec((RBL, _N), lambda i: (i, 0)),
            pl.BlockSpec((RBL, R), lambda i: (i, 0)),
        ],
        out_shape=[
            jax.ShapeDtypeStruct((_N, _N), _F32),
            jax.ShapeDtypeStruct((_N, R), _F32),
        ],
        compiler_params=_arb(1),
    )(fgp_adj, fgp_adj, rs, cs, rs, cs, s1, W2, b2r)

    zn2 = pl.pallas_call(
        _k4_body,
        grid=grid,
        in_specs=[
            pl.BlockSpec((RB, _N), lambda i: (i, 0)),
            pl.BlockSpec((_N, R), lambda i: (0, 0)),
            pl.BlockSpec((R, P), lambda i: (0, 0)),
            pl.BlockSpec((1, P), lambda i: (0, 0)),
            pl.BlockSpec((P, P), lambda i: (0, 0)),
            pl.BlockSpec((1, P), lambda i: (0, 0)),
        ],
        out_specs=pl.BlockSpec((RB, P), lambda i: (i, 0)),
        out_shape=jax.ShapeDtypeStruct((_N, P), _F32),
        compiler_params=_arb(1),
    )(an, t2, PW1, pb1r, PW2, pb2r)

    lossbuf = pl.pallas_call(
        _k5_body,
        grid=grid,
        in_specs=[
            pl.BlockSpec((RB, P), lambda i: (i, 0)),
            pl.BlockSpec((_N, P), lambda i: (0, 0)),
        ],
        out_specs=pl.BlockSpec((1, 128), lambda i: (0, 0)),
        out_shape=jax.ShapeDtypeStruct((1, 128), _F32),
        scratch_shapes=[
            pltpu.VMEM((1, _N), _F32),
            pltpu.VMEM((1, _N), _F32),
            pltpu.SMEM((1, 1), _F32),
        ],
        compiler_params=_arb(1),
    )(zn1, zn2)

    return lossbuf[0, 0], an
